# Initial kernel scaffold; baseline (speedup 1.0000x reference)
import jax
import jax.numpy as jnp
from jax.experimental import pallas as pl


def kernel(node_features, edge_index, edge_features, W1s, b1s, W2s, b2s, W3s, b3s, W1e, b1e, W2e, b2e):
    raise NotImplementedError("write your pallas kernel here")


# speedup vs baseline: 2.2233x; 2.2233x over previous
import functools

import jax
import jax.numpy as jnp
from jax import lax
from jax.experimental import pallas as pl
from jax.experimental.pallas import tpu as pltpu
from jax.experimental.pallas import tpu_sc as plsc

N = 1024
D = 128
H = 64
H2 = 32
K = 8
ED = 16

RB = 128
CB = 128

NEG_DIAG = -1e30
NEG_TAKEN = -3e38

_SC_CORES = 2
_SC_SUBCORES = 16
_SC_WORKERS = _SC_CORES * _SC_SUBCORES


def _pre_body(x_ref, w1s_ref, b1s_ref, w1et_ref, b1e_ref, w1ebp_ref,
              a_ref, b_ref, ae_ref, be_ref):
    x = x_ref[...]
    a_ref[...] = jnp.dot(x, w1s_ref[:D, :],
                         preferred_element_type=jnp.float32) + b1s_ref[...]
    b_ref[...] = jnp.dot(x, w1s_ref[D:, :],
                         preferred_element_type=jnp.float32)
    ae_ref[...] = jnp.dot(x, w1et_ref[...],
                          preferred_element_type=jnp.float32) + b1e_ref[...]
    be_ref[...] = jnp.dot(x, w1ebp_ref[...],
                          preferred_element_type=jnp.float32)


def _precompute(x, w1s, b1s, w1e_top, b1e, w1e_bot_pad):
    full = lambda shape: pl.BlockSpec(shape, lambda: (0,) * len(shape))
    out = pl.pallas_call(
        _pre_body,
        grid=(),
        in_specs=[full((N, D)), full((2 * D, H)), full((1, H)),
                  full((D, H)), full((1, H)), full((D, 2 * H))],
        out_specs=[full((N, H)), full((N, H)), full((N, H)),
                   full((N, 2 * H))],
        out_shape=[jax.ShapeDtypeStruct((N, H), jnp.float32),
                   jax.ShapeDtypeStruct((N, H), jnp.float32),
                   jax.ShapeDtypeStruct((N, H), jnp.float32),
                   jax.ShapeDtypeStruct((N, 2 * H), jnp.float32)],
    )(x, w1s, b1s, w1e_top, b1e, w1e_bot_pad)
    return out


def _sim_topk_body(a_ref, b_ref, w2_ref, b2_ref, w3_ref, b3_ref,
                   ts_ref, ti_ref):
    i = pl.program_id(0)
    a = a_ref[...]
    w2 = w2_ref[...]
    b2 = b2_ref[...]
    w3 = w3_ref[...]
    b3 = b3_ref[...]
    chunks = []
    for j in range(N // CB):
        bblk = b_ref[pl.ds(j * CB, CB), :]
        h1 = jnp.maximum(a[:, None, :] + bblk[None, :, :], 0.0)
        h1 = h1.reshape(RB * CB, H)
        h2 = jnp.maximum(
            jnp.dot(h1, w2, preferred_element_type=jnp.float32) + b2, 0.0)
        logit = jnp.dot(h2, w3, preferred_element_type=jnp.float32) + b3
        logit = logit.reshape(RB, CB)
        row_g = lax.broadcasted_iota(jnp.int32, (RB, CB), 0) + i * RB
        col_g = lax.broadcasted_iota(jnp.int32, (RB, CB), 1) + j * CB
        logit = jnp.where(row_g == col_g, NEG_DIAG, logit)
        chunks.append(logit)
    srow = jnp.concatenate(chunks, axis=1)
    colio = lax.broadcasted_iota(jnp.int32, (RB, N), 1)
    vals, idxs = [], []
    for _ in range(K):
        m = jnp.max(srow, axis=1, keepdims=True)
        idx = jnp.min(jnp.where(srow == m, colio, N), axis=1, keepdims=True)
        vals.append(m)
        idxs.append(idx)
        srow = jnp.where(colio == idx, NEG_TAKEN, srow)
    ts_ref[...] = jax.nn.sigmoid(jnp.concatenate(vals, axis=1))
    ti_ref[...] = jnp.concatenate(idxs, axis=1)


def _sim_topk(a, b, w2, b2, w3, b3):
    fixed = lambda shape: pl.BlockSpec(shape, lambda i: (0,) * len(shape))
    return pl.pallas_call(
        _sim_topk_body,
        grid=(N // RB,),
        in_specs=[pl.BlockSpec((RB, H), lambda i: (i, 0)),
                  fixed((N, H)), fixed((H, H2)), fixed((1, H2)),
                  fixed((H2, 1)), fixed((1, 1))],
        out_specs=[pl.BlockSpec((RB, K), lambda i: (i, 0)),
                   pl.BlockSpec((RB, K), lambda i: (i, 0))],
        out_shape=[jax.ShapeDtypeStruct((N, K), jnp.float32),
                   jax.ShapeDtypeStruct((N, K), jnp.int32)],
        compiler_params=pltpu.CompilerParams(
            dimension_semantics=("arbitrary",)),
    )(a, b, w2, b2, w3, b3)


_GB = (N * K) // _SC_WORKERS


def _sc_gather_body(table_hbm, idx_hbm, out_hbm, idx_v, rows_v, sem):
    wid = lax.axis_index("s") * _SC_CORES + lax.axis_index("c")
    base = wid * _GB
    pltpu.sync_copy(idx_hbm.at[pl.ds(base, _GB)], idx_v)
    pltpu.async_copy(table_hbm.at[idx_v], rows_v, sem).wait()
    pltpu.sync_copy(rows_v, out_hbm.at[pl.ds(base, _GB)])


_sc_gather_fn = None


def _sc_gather(table, idx):
    global _sc_gather_fn
    if _sc_gather_fn is None:
        _sc_gather_fn = pl.kernel(
            _sc_gather_body,
            out_type=jax.ShapeDtypeStruct((N * K, 2 * H), jnp.float32),
            mesh=plsc.VectorSubcoreMesh(core_axis_name="c",
                                        subcore_axis_name="s"),
            scratch_types=[pltpu.VMEM((_GB,), jnp.int32),
                           pltpu.VMEM((_GB, 2 * H), jnp.float32),
                           pltpu.SemaphoreType.DMA],
        )
    return _sc_gather_fn(table, idx)


def _edge_body(ae_ref, beg_ref, w2e_ref, b2e_ref, ts_ref, out_ref):
    ae = ae_ref[...]
    ae_rep = jnp.broadcast_to(ae[:, None, :], (N, K, H)).reshape(N * K, H)
    he = jnp.maximum(ae_rep + beg_ref[:, :H], 0.0)
    ef = jnp.dot(he, w2e_ref[...],
                 preferred_element_type=jnp.float32) + b2e_ref[...]
    ts = ts_ref[...]
    out_ref[:, 0:1] = ts
    out_ref[:, 1:1 + ED] = jnp.where(ts >= 0.5, ef, 0.0)


def _edge(ae, beg, w2e, b2e, ts_flat):
    full = lambda shape: pl.BlockSpec(shape, lambda: (0,) * len(shape))
    return pl.pallas_call(
        _edge_body,
        grid=(),
        in_specs=[full((N, H)), full((N * K, 2 * H)), full((H, ED)),
                  full((1, ED)), full((N * K, 1))],
        out_specs=full((N * K, 1 + ED)),
        out_shape=jax.ShapeDtypeStruct((N * K, 1 + ED), jnp.float32),
    )(ae, beg, w2e, b2e, ts_flat)


def kernel(node_features, edge_index, edge_features,
           W1s, b1s, W2s, b2s, W3s, b3s, W1e, b1e, W2e, b2e):
    del edge_index, edge_features
    w1e_bot_pad = jnp.concatenate(
        [W1e[D:], jnp.zeros((D, H), jnp.float32)], axis=1)
    a, b, ae, be = _precompute(node_features, W1s, b1s.reshape(1, H),
                               W1e[:D], b1e.reshape(1, H), w1e_bot_pad)
    ts, ti = _sim_topk(a, b, W2s, b2s.reshape(1, H2), W3s, b3s.reshape(1, 1))
    beg = _sc_gather(be, ti.reshape(-1))
    return _edge(ae, beg, W2e, b2e.reshape(1, ED), ts.reshape(N * K, 1))
